# baseline (speedup 1.0000x reference)
import functools

import numpy as np

import jax
import jax.numpy as jnp
from jax import lax
from jax.experimental import pallas as pl
from jax.experimental.pallas import tpu as pltpu
from jax.experimental.pallas import tpu_sc as plsc

_N = 10000
_E = 160000
_D = 256
_HALF = 128
_NPAD = 10240
_CHUNK = 128
_NCHUNK = 80
_GRP = 8
_NGRP = _NCHUNK // _GRP
_EPT = _CHUNK * _NCHUNK
_EPAD = _EPT * 16
_RPT = _NPAD // 16
_PAD_SRC = np.asarray((np.arange(_EPAD - _E) * 37) % _N, np.int32)
_PAD_DST = np.asarray(_N + np.arange(_EPAD - _E) % (_NPAD - _N), np.int32)
_BM = 512
_BM2 = 1024


def _make_agg(with_counts):
    mesh = plsc.VectorSubcoreMesh(core_axis_name="c", subcore_axis_name="s")

    out_type = [
        jax.ShapeDtypeStruct((_NPAD, _HALF), jnp.float32),
        jax.ShapeDtypeStruct((_NPAD, _HALF), jnp.float32),
    ]
    scratch = [
        pltpu.VMEM((_GRP, _CHUNK), jnp.int32),
        pltpu.VMEM((_GRP, _CHUNK), jnp.int32),
        pltpu.VMEM((_GRP, _CHUNK), jnp.int32),
        pltpu.VMEM((_GRP, _CHUNK), jnp.int32),
        pltpu.VMEM((_CHUNK, _HALF), jnp.float32),
        pltpu.VMEM((_CHUNK, _HALF), jnp.float32),
        pltpu.VMEM_SHARED((_NPAD, _HALF), jnp.float32),
        pltpu.SemaphoreType.DMA,
        pltpu.SemaphoreType.DMA,
        pltpu.SemaphoreType.DMA,
        pltpu.SemaphoreType.DMA,
        pltpu.SemaphoreType.DMA,
    ]
    if with_counts:
        out_type.append(jax.ShapeDtypeStruct((_NPAD,), jnp.float32))
        scratch += [
            pltpu.VMEM((_CHUNK,), jnp.float32),
            pltpu.VMEM_SHARED((_NPAD,), jnp.float32),
            pltpu.SemaphoreType.DMA,
        ]

    @functools.partial(pl.kernel, out_type=tuple(out_type), mesh=mesh,
                       scratch_types=scratch)
    def agg(x0, x1, srcg, dstg, z2, *rest):
        if with_counts:
            (z1, s0, s1, cnt,
             si0, di0, si1, di1, rows0, rows1, acc,
             g0, g1, t0, t1, isem, ones, acc1, osem) = rest
        else:
            (s0, s1,
             si0, di0, si1, di1, rows0, rows1, acc,
             g0, g1, t0, t1, isem) = rest
        c = lax.axis_index("c")
        s = lax.axis_index("s")
        base = s * _RPT

        rows = (rows0, rows1)
        gsem = (g0, g1)
        ssem = (t0, t1)
        src_i = (si0, si1)
        dst_i = (di0, di1)

        def start_gather(idx_row, buf, sem):
            @pl.when(c == 0)
            def _():
                pltpu.async_copy(x0.at[idx_row], buf, sem)

            @pl.when(c == 1)
            def _():
                pltpu.async_copy(x1.at[idx_row], buf, sem)

        def drain(buf, sem):
            pltpu.make_async_copy(x0.at[si0.at[0]], buf, sem).wait()

        def stage_idx(g, cur, sem):
            base_row = s * _NCHUNK + g * _GRP
            pltpu.async_copy(srcg.at[pl.ds(base_row, _GRP)], src_i[cur], sem)
            pltpu.async_copy(dstg.at[pl.ds(base_row, _GRP)], dst_i[cur], sem)

        def wait_idx(cur):
            pltpu.make_async_copy(srcg.at[pl.ds(0, _GRP)], src_i[cur], isem).wait()
            pltpu.make_async_copy(dstg.at[pl.ds(0, _GRP)], dst_i[cur], isem).wait()

        stage_idx(0, 0, isem)
        pltpu.sync_copy(z2, acc.at[pl.ds(base, _RPT)])
        if with_counts:
            @pl.when(c == 0)
            def _():
                pltpu.sync_copy(z1, acc1.at[pl.ds(base, _RPT)])

            for i in range(_CHUNK // 16):
                ones[pl.ds(i * 16, 16)] = jnp.ones((16,), jnp.float32)
        wait_idx(0)
        start_gather(si0.at[0], rows[0], gsem[0])
        plsc.subcore_barrier()

        def drain_ones():
            if with_counts:
                @pl.when(c == 0)
                def _():
                    for _i in range(_GRP):
                        pltpu.make_async_copy(
                            ones, acc1.at[di0.at[0]], osem).wait()

        def do_group(g, cur, first_pred, next_pred):
            if first_pred is None:
                drain_ones()
            else:
                @pl.when(first_pred)
                def _():
                    drain_ones()
            if next_pred is None:
                stage_idx(g + 1, 1 - cur, isem)
            else:
                @pl.when(next_pred)
                def _():
                    stage_idx(g + 1, 1 - cur, isem)

            for b in range(_GRP):
                k = b % 2
                nk = 1 - k
                if b + 1 < _GRP:
                    if b == 0 and first_pred is not None:
                        @pl.when(first_pred)
                        def _():
                            drain(rows[nk], ssem[nk])
                    else:
                        drain(rows[nk], ssem[nk])

                    start_gather(src_i[cur].at[b + 1], rows[nk], gsem[nk])
                else:
                    def boundary():
                        drain(rows[nk], ssem[nk])
                        wait_idx(1 - cur)
                        start_gather(src_i[1 - cur].at[0], rows[nk], gsem[nk])

                    if next_pred is None:
                        boundary()
                    else:
                        pl.when(next_pred)(boundary)

                drain(rows[k], gsem[k])
                pltpu.async_copy(rows[k], acc.at[dst_i[cur].at[b]], ssem[k],
                                 add=True)
                if with_counts:
                    @pl.when(c == 0)
                    def _():
                        pltpu.async_copy(ones, acc1.at[dst_i[cur].at[b]],
                                         osem, add=True)

        def pair(gp, carry):
            do_group(2 * gp, 0, gp > 0, None)
            do_group(2 * gp + 1, 1, None, gp < _NGRP // 2 - 1)
            return carry

        lax.fori_loop(0, _NGRP // 2, pair, 0)
        drain(rows[0], ssem[0])
        drain(rows[1], ssem[1])
        drain_ones()

        plsc.subcore_barrier()

        @pl.when(c == 0)
        def _():
            pltpu.sync_copy(acc.at[pl.ds(base, _RPT)], s0.at[pl.ds(base, _RPT)])
            if with_counts:
                pltpu.sync_copy(acc1.at[pl.ds(base, _RPT)],
                                cnt.at[pl.ds(base, _RPT)])

        @pl.when(c == 1)
        def _():
            pltpu.sync_copy(acc.at[pl.ds(base, _RPT)], s1.at[pl.ds(base, _RPT)])

    return agg


_agg_c = _make_agg(True)
_agg_nc = _make_agg(False)


def _xwr(h0, h1, wrt, b):
    def body(x0_r, x1_r, wr_r, b_r, o_r):
        wr = wr_r[...].astype(jnp.bfloat16)
        o_r[...] = (jnp.dot(x0_r[...].astype(jnp.bfloat16), wr[:_HALF],
                            preferred_element_type=jnp.float32)
                    + jnp.dot(x1_r[...].astype(jnp.bfloat16), wr[_HALF:],
                              preferred_element_type=jnp.float32)
                    + b_r[...]).astype(jnp.bfloat16)

    row_block = lambda w: pl.BlockSpec((_BM, w), lambda i: (i, 0))
    whole = lambda a: pl.BlockSpec(a.shape, lambda i: (0, 0))
    return pl.pallas_call(
        body,
        grid=(pl.cdiv(_N, _BM),),
        in_specs=[row_block(_HALF), row_block(_HALF), whole(wrt), whole(b)],
        out_specs=row_block(_D),
        out_shape=jax.ShapeDtypeStruct((_N, _D), jnp.bfloat16),
    )(h0, h1, wrt, b)


def _combine(s0, s1, cnt, xwr, wlt, *, relu, full):
    def body(s0_r, s1_r, c_r, xw_r, wl_r, *outs):
        inv = 1.0 / jnp.maximum(c_r[...], 1.0)
        m0 = (s0_r[...] * inv).astype(jnp.bfloat16)
        m1 = (s1_r[...] * inv).astype(jnp.bfloat16)
        wl = wl_r[...].astype(jnp.bfloat16)
        a = (jnp.dot(m0, wl[:_HALF], preferred_element_type=jnp.float32)
             + jnp.dot(m1, wl[_HALF:], preferred_element_type=jnp.float32)
             + xw_r[...].astype(jnp.float32))
        if relu:
            a = jnp.maximum(a, 0.0)
        if full:
            outs[0][...] = a
        else:
            outs[0][...] = a[:, :_HALF]
            outs[1][...] = a[:, _HALF:]

    row_block = lambda w: pl.BlockSpec((_BM2, w), lambda i: (i, 0))
    whole = lambda a: pl.BlockSpec(a.shape, lambda i: (0, 0))
    if full:
        out_shape = (jax.ShapeDtypeStruct((_N, _D), jnp.float32),)
        out_specs = (row_block(_D),)
    else:
        out_shape = (jax.ShapeDtypeStruct((_N, _HALF), jnp.float32),) * 2
        out_specs = (row_block(_HALF), row_block(_HALF))
    return pl.pallas_call(
        body,
        grid=(_NPAD // _BM2,),
        in_specs=[row_block(_HALF), row_block(_HALF), row_block(1),
                  row_block(_D), whole(wlt)],
        out_specs=out_specs,
        out_shape=out_shape,
    )(s0, s1, cnt, xwr, wlt)


def kernel(x, adj, Wl1, Wr1, b1, Wl2, Wr2, b2, Wl3, Wr3, b3):
    src = adj[0]
    dst = adj[1]
    srcg = jnp.concatenate([src, jnp.asarray(_PAD_SRC)]).reshape(
        _EPAD // _CHUNK, _CHUNK)
    dstg = jnp.concatenate([dst, jnp.asarray(_PAD_DST)]).reshape(
        _EPAD // _CHUNK, _CHUNK)
    z2 = jnp.zeros((_RPT, _HALF), jnp.float32)
    z1 = jnp.zeros((_RPT,), jnp.float32)

    h0 = x[:, :_HALF]
    h1 = x[:, _HALF:]

    def layer(h0, h1, cnt, Wl, Wr, b, relu, full):
        xwr = _xwr(h0, h1, Wr.T, b.reshape(1, _D))
        if cnt is None:
            s0, s1, cnt = _agg_c(h0, h1, srcg, dstg, z2, z1)
            cnt = cnt.reshape(_NPAD, 1)
        else:
            s0, s1 = _agg_nc(h0, h1, srcg, dstg, z2)
        res = _combine(s0, s1, cnt, xwr, Wl.T, relu=relu, full=full)
        if full:
            return (res if isinstance(res, jax.Array) else res[0],) + (cnt,)
        return tuple(res) + (cnt,)

    h0, h1, cnt = layer(h0, h1, None, Wl1, Wr1, b1, True, False)
    h0, h1, cnt = layer(h0, h1, cnt, Wl2, Wr2, b2, True, False)
    (out, _) = layer(h0, h1, cnt, Wl3, Wr3, b3, False, True)
    return out


# speedup vs baseline: 1.0133x; 1.0133x over previous
import functools

import numpy as np

import jax
import jax.numpy as jnp
from jax import lax
from jax.experimental import pallas as pl
from jax.experimental.pallas import tpu as pltpu
from jax.experimental.pallas import tpu_sc as plsc

_N = 10000
_E = 160000
_D = 256
_HALF = 128
_NPAD = 10240
_CHUNK = 128
_NCHUNK = 80
_GRP = 8
_NGRP = _NCHUNK // _GRP
_EPT = _CHUNK * _NCHUNK
_EPAD = _EPT * 16
_RPT = _NPAD // 16
_PAD_SRC = np.asarray((np.arange(_EPAD - _E) * 37) % _N, np.int32)
_PAD_DST = np.asarray(_N + np.arange(_EPAD - _E) % (_NPAD - _N), np.int32)
_BM = 512
_BM2 = 1024


def _make_agg(with_counts):
    mesh = plsc.VectorSubcoreMesh(core_axis_name="c", subcore_axis_name="s")

    out_type = [
        jax.ShapeDtypeStruct((_NPAD, _HALF), jnp.float32),
        jax.ShapeDtypeStruct((_NPAD, _HALF), jnp.float32),
    ]
    scratch = [
        pltpu.VMEM((_GRP, _CHUNK), jnp.int32),
        pltpu.VMEM((_GRP, _CHUNK), jnp.int32),
        pltpu.VMEM((_GRP, _CHUNK), jnp.int32),
        pltpu.VMEM((_GRP, _CHUNK), jnp.int32),
        pltpu.VMEM((_CHUNK, _HALF), jnp.float32),
        pltpu.VMEM((_CHUNK, _HALF), jnp.float32),
        pltpu.VMEM_SHARED((_NPAD, _HALF), jnp.float32),
        pltpu.SemaphoreType.DMA,
        pltpu.SemaphoreType.DMA,
        pltpu.SemaphoreType.DMA,
        pltpu.SemaphoreType.DMA,
        pltpu.SemaphoreType.DMA,
    ]
    if with_counts:
        out_type.append(jax.ShapeDtypeStruct((_NPAD,), jnp.float32))
        scratch += [
            pltpu.VMEM((_CHUNK,), jnp.float32),
            pltpu.VMEM_SHARED((_NPAD,), jnp.float32),
            pltpu.SemaphoreType.DMA,
        ]

    @functools.partial(pl.kernel, out_type=tuple(out_type), mesh=mesh,
                       scratch_types=scratch)
    def agg(x0, x1, srcg, dstg, z2, *rest):
        if with_counts:
            (z1, s0, s1, cnt,
             si0, di0, si1, di1, rows0, rows1, acc,
             g0, g1, t0, t1, isem, ones, acc1, osem) = rest
        else:
            (s0, s1,
             si0, di0, si1, di1, rows0, rows1, acc,
             g0, g1, t0, t1, isem) = rest
        c = lax.axis_index("c")
        s = lax.axis_index("s")
        base = s * _RPT

        rows = (rows0, rows1)
        gsem = (g0, g1)
        ssem = (t0, t1)
        src_i = (si0, si1)
        dst_i = (di0, di1)

        def start_gather(idx_row, buf, sem):
            @pl.when(c == 0)
            def _():
                pltpu.async_copy(x0.at[idx_row], buf, sem)

            @pl.when(c == 1)
            def _():
                pltpu.async_copy(x1.at[idx_row], buf, sem)

        def drain(buf, sem):
            pltpu.make_async_copy(x0.at[si0.at[0]], buf, sem).wait()

        def stage_idx(g, cur, sem):
            base_row = s * _NCHUNK + g * _GRP
            pltpu.async_copy(srcg.at[pl.ds(base_row, _GRP)], src_i[cur], sem)
            pltpu.async_copy(dstg.at[pl.ds(base_row, _GRP)], dst_i[cur], sem)

        def wait_idx(cur):
            pltpu.make_async_copy(srcg.at[pl.ds(0, _GRP)], src_i[cur], isem).wait()
            pltpu.make_async_copy(dstg.at[pl.ds(0, _GRP)], dst_i[cur], isem).wait()

        stage_idx(0, 0, isem)
        pltpu.sync_copy(z2.at[pl.ds(base, _RPT)], acc.at[pl.ds(base, _RPT)])
        if with_counts:
            @pl.when(c == 0)
            def _():
                pltpu.sync_copy(z1.at[pl.ds(base, _RPT)],
                                acc1.at[pl.ds(base, _RPT)])

            for i in range(_CHUNK // 16):
                ones[pl.ds(i * 16, 16)] = jnp.ones((16,), jnp.float32)
        wait_idx(0)
        start_gather(si0.at[0], rows[0], gsem[0])
        plsc.subcore_barrier()

        def drain_ones():
            if with_counts:
                @pl.when(c == 0)
                def _():
                    for _i in range(_GRP):
                        pltpu.make_async_copy(
                            ones, acc1.at[di0.at[0]], osem).wait()

        def do_group(g, cur, first_pred, next_pred):
            if first_pred is None:
                drain_ones()
            else:
                @pl.when(first_pred)
                def _():
                    drain_ones()
            if next_pred is None:
                stage_idx(g + 1, 1 - cur, isem)
            else:
                @pl.when(next_pred)
                def _():
                    stage_idx(g + 1, 1 - cur, isem)

            for b in range(_GRP):
                k = b % 2
                nk = 1 - k
                if b + 1 < _GRP:
                    if b == 0 and first_pred is not None:
                        @pl.when(first_pred)
                        def _():
                            drain(rows[nk], ssem[nk])
                    else:
                        drain(rows[nk], ssem[nk])

                    start_gather(src_i[cur].at[b + 1], rows[nk], gsem[nk])
                else:
                    def boundary():
                        drain(rows[nk], ssem[nk])
                        wait_idx(1 - cur)
                        start_gather(src_i[1 - cur].at[0], rows[nk], gsem[nk])

                    if next_pred is None:
                        boundary()
                    else:
                        pl.when(next_pred)(boundary)

                drain(rows[k], gsem[k])
                pltpu.async_copy(rows[k], acc.at[dst_i[cur].at[b]], ssem[k],
                                 add=True)
                if with_counts:
                    @pl.when(c == 0)
                    def _():
                        pltpu.async_copy(ones, acc1.at[dst_i[cur].at[b]],
                                         osem, add=True)

        def pair(gp, carry):
            do_group(2 * gp, 0, gp > 0, None)
            do_group(2 * gp + 1, 1, None, gp < _NGRP // 2 - 1)
            return carry

        lax.fori_loop(0, _NGRP // 2, pair, 0)
        drain(rows[0], ssem[0])
        drain(rows[1], ssem[1])
        drain_ones()

        plsc.subcore_barrier()

        @pl.when(c == 0)
        def _():
            pltpu.sync_copy(acc.at[pl.ds(base, _RPT)], s0.at[pl.ds(base, _RPT)])
            if with_counts:
                pltpu.sync_copy(acc1.at[pl.ds(base, _RPT)],
                                cnt.at[pl.ds(base, _RPT)])

        @pl.when(c == 1)
        def _():
            pltpu.sync_copy(acc.at[pl.ds(base, _RPT)], s1.at[pl.ds(base, _RPT)])

    return agg


_agg_c = _make_agg(True)
_agg_nc = _make_agg(False)


def _xwr(h0, h1, wrt, b):
    def body(x0_r, x1_r, wr_r, b_r, o_r):
        wr = wr_r[...].astype(jnp.bfloat16)
        o_r[...] = (jnp.dot(x0_r[...].astype(jnp.bfloat16), wr[:_HALF],
                            preferred_element_type=jnp.float32)
                    + jnp.dot(x1_r[...].astype(jnp.bfloat16), wr[_HALF:],
                              preferred_element_type=jnp.float32)
                    + b_r[...]).astype(jnp.bfloat16)

    row_block = lambda w: pl.BlockSpec((_BM, w), lambda i: (i, 0))
    whole = lambda a: pl.BlockSpec(a.shape, lambda i: (0, 0))
    return pl.pallas_call(
        body,
        grid=(pl.cdiv(_N, _BM),),
        in_specs=[row_block(_HALF), row_block(_HALF), whole(wrt), whole(b)],
        out_specs=row_block(_D),
        out_shape=jax.ShapeDtypeStruct((_N, _D), jnp.bfloat16),
    )(h0, h1, wrt, b)


def _combine(s0, s1, cnt, xwr, wlt, *, relu, full):
    def body(s0_r, s1_r, c_r, xw_r, wl_r, *outs):
        inv = 1.0 / jnp.maximum(c_r[...], 1.0)
        m0 = (s0_r[...] * inv).astype(jnp.bfloat16)
        m1 = (s1_r[...] * inv).astype(jnp.bfloat16)
        wl = wl_r[...].astype(jnp.bfloat16)
        a = (jnp.dot(m0, wl[:_HALF], preferred_element_type=jnp.float32)
             + jnp.dot(m1, wl[_HALF:], preferred_element_type=jnp.float32)
             + xw_r[...].astype(jnp.float32))
        if relu:
            a = jnp.maximum(a, 0.0)
        if full:
            outs[0][...] = a
        else:
            outs[0][...] = a[:, :_HALF]
            outs[1][...] = a[:, _HALF:]

    row_block = lambda w: pl.BlockSpec((_BM2, w), lambda i: (i, 0))
    whole = lambda a: pl.BlockSpec(a.shape, lambda i: (0, 0))
    if full:
        out_shape = (jax.ShapeDtypeStruct((_N, _D), jnp.float32),)
        out_specs = (row_block(_D),)
    else:
        out_shape = (jax.ShapeDtypeStruct((_N, _HALF), jnp.float32),) * 2
        out_specs = (row_block(_HALF), row_block(_HALF))
    return pl.pallas_call(
        body,
        grid=(_NPAD // _BM2,),
        in_specs=[row_block(_HALF), row_block(_HALF), row_block(1),
                  row_block(_D), whole(wlt)],
        out_specs=out_specs,
        out_shape=out_shape,
    )(s0, s1, cnt, xwr, wlt)


def kernel(x, adj, Wl1, Wr1, b1, Wl2, Wr2, b2, Wl3, Wr3, b3):
    src = adj[0]
    dst = adj[1]
    srcg = jnp.concatenate([src, jnp.asarray(_PAD_SRC)]).reshape(
        _EPAD // _CHUNK, _CHUNK)
    dstg = jnp.concatenate([dst, jnp.asarray(_PAD_DST)]).reshape(
        _EPAD // _CHUNK, _CHUNK)
    z2 = jnp.zeros((_NPAD, _HALF), jnp.float32)
    z1 = jnp.zeros((_NPAD,), jnp.float32)

    h0 = x[:, :_HALF]
    h1 = x[:, _HALF:]

    def layer(h0, h1, cnt, Wl, Wr, b, relu, full):
        xwr = _xwr(h0, h1, Wr.T, b.reshape(1, _D))
        if cnt is None:
            s0, s1, cnt = _agg_c(h0, h1, srcg, dstg, z2, z1)
            cnt = cnt.reshape(_NPAD, 1)
        else:
            s0, s1 = _agg_nc(h0, h1, srcg, dstg, z2)
        res = _combine(s0, s1, cnt, xwr, Wl.T, relu=relu, full=full)
        if full:
            return (res if isinstance(res, jax.Array) else res[0],) + (cnt,)
        return tuple(res) + (cnt,)

    h0, h1, cnt = layer(h0, h1, None, Wl1, Wr1, b1, True, False)
    h0, h1, cnt = layer(h0, h1, cnt, Wl2, Wr2, b2, True, False)
    (out, _) = layer(h0, h1, cnt, Wl3, Wr3, b3, False, True)
    return out


# speedup vs baseline: 1.0227x; 1.0093x over previous
import functools

import numpy as np

import jax
import jax.numpy as jnp
from jax import lax
from jax.experimental import pallas as pl
from jax.experimental.pallas import tpu as pltpu
from jax.experimental.pallas import tpu_sc as plsc

_N = 10000
_E = 160000
_D = 256
_HALF = 128
_NPAD = 10240
_CHUNK = 128
_NCHUNK = 80
_GRP = 8
_NGRP = _NCHUNK // _GRP
_EPT = _CHUNK * _NCHUNK
_EPAD = _EPT * 16
_RPT = _NPAD // 16
_PAD_SRC = np.asarray((np.arange(_EPAD - _E) * 37) % _N, np.int32)
_PAD_DST = np.asarray(_N + np.arange(_EPAD - _E) % (_NPAD - _N), np.int32)
_BM = 512
_BM2 = 2048


def _make_agg(with_counts):
    mesh = plsc.VectorSubcoreMesh(core_axis_name="c", subcore_axis_name="s")

    out_type = [
        jax.ShapeDtypeStruct((_NPAD, _HALF), jnp.float32),
        jax.ShapeDtypeStruct((_NPAD, _HALF), jnp.float32),
    ]
    scratch = [
        pltpu.VMEM((_GRP, _CHUNK), jnp.int32),
        pltpu.VMEM((_GRP, _CHUNK), jnp.int32),
        pltpu.VMEM((_GRP, _CHUNK), jnp.int32),
        pltpu.VMEM((_GRP, _CHUNK), jnp.int32),
        pltpu.VMEM((_CHUNK, _HALF), jnp.float32),
        pltpu.VMEM((_CHUNK, _HALF), jnp.float32),
        pltpu.VMEM_SHARED((_NPAD, _HALF), jnp.float32),
        pltpu.SemaphoreType.DMA,
        pltpu.SemaphoreType.DMA,
        pltpu.SemaphoreType.DMA,
        pltpu.SemaphoreType.DMA,
        pltpu.SemaphoreType.DMA,
    ]
    if with_counts:
        out_type.append(jax.ShapeDtypeStruct((_NPAD,), jnp.float32))
        scratch += [
            pltpu.VMEM((_CHUNK,), jnp.float32),
            pltpu.VMEM_SHARED((_NPAD,), jnp.float32),
            pltpu.SemaphoreType.DMA,
        ]

    @functools.partial(pl.kernel, out_type=tuple(out_type), mesh=mesh,
                       scratch_types=scratch)
    def agg(x0, x1, srcg, dstg, z2, *rest):
        if with_counts:
            (z1, s0, s1, cnt,
             si0, di0, si1, di1, rows0, rows1, acc,
             g0, g1, t0, t1, isem, ones, acc1, osem) = rest
        else:
            (s0, s1,
             si0, di0, si1, di1, rows0, rows1, acc,
             g0, g1, t0, t1, isem) = rest
        c = lax.axis_index("c")
        s = lax.axis_index("s")
        base = s * _RPT

        rows = (rows0, rows1)
        gsem = (g0, g1)
        ssem = (t0, t1)
        src_i = (si0, si1)
        dst_i = (di0, di1)

        def start_gather(idx_row, buf, sem):
            @pl.when(c == 0)
            def _():
                pltpu.async_copy(x0.at[idx_row], buf, sem)

            @pl.when(c == 1)
            def _():
                pltpu.async_copy(x1.at[idx_row], buf, sem)

        def drain(buf, sem):
            pltpu.make_async_copy(x0.at[si0.at[0]], buf, sem).wait()

        def stage_idx(g, cur, sem):
            base_row = s * _NCHUNK + g * _GRP
            pltpu.async_copy(srcg.at[pl.ds(base_row, _GRP)], src_i[cur], sem)
            pltpu.async_copy(dstg.at[pl.ds(base_row, _GRP)], dst_i[cur], sem)

        def wait_idx(cur):
            pltpu.make_async_copy(srcg.at[pl.ds(0, _GRP)], src_i[cur], isem).wait()
            pltpu.make_async_copy(dstg.at[pl.ds(0, _GRP)], dst_i[cur], isem).wait()

        stage_idx(0, 0, isem)
        pltpu.sync_copy(z2.at[pl.ds(base, _RPT)], acc.at[pl.ds(base, _RPT)])
        if with_counts:
            @pl.when(c == 0)
            def _():
                pltpu.sync_copy(z1.at[pl.ds(base, _RPT)],
                                acc1.at[pl.ds(base, _RPT)])

            for i in range(_CHUNK // 16):
                ones[pl.ds(i * 16, 16)] = jnp.ones((16,), jnp.float32)
        wait_idx(0)
        start_gather(si0.at[0], rows[0], gsem[0])
        plsc.subcore_barrier()

        def drain_ones():
            if with_counts:
                @pl.when(c == 0)
                def _():
                    for _i in range(_GRP):
                        pltpu.make_async_copy(
                            ones, acc1.at[di0.at[0]], osem).wait()

        def do_group(g, cur, first_pred, next_pred):
            if first_pred is None:
                drain_ones()
            else:
                @pl.when(first_pred)
                def _():
                    drain_ones()
            if next_pred is None:
                stage_idx(g + 1, 1 - cur, isem)
            else:
                @pl.when(next_pred)
                def _():
                    stage_idx(g + 1, 1 - cur, isem)

            for b in range(_GRP):
                k = b % 2
                nk = 1 - k
                if b + 1 < _GRP:
                    if b == 0 and first_pred is not None:
                        @pl.when(first_pred)
                        def _():
                            drain(rows[nk], ssem[nk])
                    else:
                        drain(rows[nk], ssem[nk])

                    start_gather(src_i[cur].at[b + 1], rows[nk], gsem[nk])
                else:
                    def boundary():
                        drain(rows[nk], ssem[nk])
                        wait_idx(1 - cur)
                        start_gather(src_i[1 - cur].at[0], rows[nk], gsem[nk])

                    if next_pred is None:
                        boundary()
                    else:
                        pl.when(next_pred)(boundary)

                drain(rows[k], gsem[k])
                pltpu.async_copy(rows[k], acc.at[dst_i[cur].at[b]], ssem[k],
                                 add=True)
                if with_counts:
                    @pl.when(c == 0)
                    def _():
                        pltpu.async_copy(ones, acc1.at[dst_i[cur].at[b]],
                                         osem, add=True)

        def pair(gp, carry):
            do_group(2 * gp, 0, gp > 0, None)
            do_group(2 * gp + 1, 1, None, gp < _NGRP // 2 - 1)
            return carry

        lax.fori_loop(0, _NGRP // 2, pair, 0)
        drain(rows[0], ssem[0])
        drain(rows[1], ssem[1])
        drain_ones()

        plsc.subcore_barrier()

        @pl.when(c == 0)
        def _():
            pltpu.sync_copy(acc.at[pl.ds(base, _RPT)], s0.at[pl.ds(base, _RPT)])
            if with_counts:
                pltpu.sync_copy(acc1.at[pl.ds(base, _RPT)],
                                cnt.at[pl.ds(base, _RPT)])

        @pl.when(c == 1)
        def _():
            pltpu.sync_copy(acc.at[pl.ds(base, _RPT)], s1.at[pl.ds(base, _RPT)])

    return agg


_agg_c = _make_agg(True)
_agg_nc = _make_agg(False)


def _xwr(h0, h1, wrt, b):
    def body(x0_r, x1_r, wr_r, b_r, o_r):
        wr = wr_r[...].astype(jnp.bfloat16)
        o_r[...] = (jnp.dot(x0_r[...].astype(jnp.bfloat16), wr[:_HALF],
                            preferred_element_type=jnp.float32)
                    + jnp.dot(x1_r[...].astype(jnp.bfloat16), wr[_HALF:],
                              preferred_element_type=jnp.float32)
                    + b_r[...]).astype(jnp.bfloat16)

    row_block = lambda w: pl.BlockSpec((_BM, w), lambda i: (i, 0))
    whole = lambda a: pl.BlockSpec(a.shape, lambda i: (0, 0))
    return pl.pallas_call(
        body,
        grid=(pl.cdiv(_N, _BM),),
        in_specs=[row_block(_HALF), row_block(_HALF), whole(wrt), whole(b)],
        out_specs=row_block(_D),
        out_shape=jax.ShapeDtypeStruct((_N, _D), jnp.bfloat16),
    )(h0, h1, wrt, b)


def _combine(s0, s1, cnt, xwr, wlt, *, relu, full):
    def body(s0_r, s1_r, c_r, xw_r, wl_r, *outs):
        inv = 1.0 / jnp.maximum(c_r[...], 1.0)
        m0 = (s0_r[...] * inv).astype(jnp.bfloat16)
        m1 = (s1_r[...] * inv).astype(jnp.bfloat16)
        wl = wl_r[...].astype(jnp.bfloat16)
        a = (jnp.dot(m0, wl[:_HALF], preferred_element_type=jnp.float32)
             + jnp.dot(m1, wl[_HALF:], preferred_element_type=jnp.float32)
             + xw_r[...].astype(jnp.float32))
        if relu:
            a = jnp.maximum(a, 0.0)
        if full:
            outs[0][...] = a
        else:
            outs[0][...] = a[:, :_HALF]
            outs[1][...] = a[:, _HALF:]

    row_block = lambda w: pl.BlockSpec((_BM2, w), lambda i: (i, 0))
    whole = lambda a: pl.BlockSpec(a.shape, lambda i: (0, 0))
    if full:
        out_shape = (jax.ShapeDtypeStruct((_N, _D), jnp.float32),)
        out_specs = (row_block(_D),)
    else:
        out_shape = (jax.ShapeDtypeStruct((_N, _HALF), jnp.float32),) * 2
        out_specs = (row_block(_HALF), row_block(_HALF))
    return pl.pallas_call(
        body,
        grid=(_NPAD // _BM2,),
        in_specs=[row_block(_HALF), row_block(_HALF), row_block(1),
                  row_block(_D), whole(wlt)],
        out_specs=out_specs,
        out_shape=out_shape,
    )(s0, s1, cnt, xwr, wlt)


def kernel(x, adj, Wl1, Wr1, b1, Wl2, Wr2, b2, Wl3, Wr3, b3):
    src = adj[0]
    dst = adj[1]
    srcg = jnp.concatenate([src, jnp.asarray(_PAD_SRC)]).reshape(
        _EPAD // _CHUNK, _CHUNK)
    dstg = jnp.concatenate([dst, jnp.asarray(_PAD_DST)]).reshape(
        _EPAD // _CHUNK, _CHUNK)
    z2 = jnp.zeros((_NPAD, _HALF), jnp.float32)
    z1 = jnp.zeros((_NPAD,), jnp.float32)

    h0 = x[:, :_HALF]
    h1 = x[:, _HALF:]

    def layer(h0, h1, cnt, Wl, Wr, b, relu, full):
        xwr = _xwr(h0, h1, Wr.T, b.reshape(1, _D))
        if cnt is None:
            s0, s1, cnt = _agg_c(h0, h1, srcg, dstg, z2, z1)
            cnt = cnt.reshape(_NPAD, 1)
        else:
            s0, s1 = _agg_nc(h0, h1, srcg, dstg, z2)
        res = _combine(s0, s1, cnt, xwr, Wl.T, relu=relu, full=full)
        if full:
            return (res if isinstance(res, jax.Array) else res[0],) + (cnt,)
        return tuple(res) + (cnt,)

    h0, h1, cnt = layer(h0, h1, None, Wl1, Wr1, b1, True, False)
    h0, h1, cnt = layer(h0, h1, cnt, Wl2, Wr2, b2, True, False)
    (out, _) = layer(h0, h1, cnt, Wl3, Wr3, b3, False, True)
    return out
